# baseline (speedup 1.0000x reference)
import functools
import math

import jax
import jax.numpy as jnp
from jax import lax
from jax.experimental import pallas as pl
from jax.experimental.pallas import tpu as pltpu
from jax.experimental.pallas import tpu_sc as plsc

B, NB, NS, C, H, TOPK = 8, 576, 1024, 768, 12, 128
DH = C // H
HID = 4 * C
NV = NS // 16


def _scores_body(sampled_ref, base_ref, out_ref):
    g = pl.program_id(0)
    for bb in range(2):
        s = sampled_ref[bb]
        bt = base_ref[bb]
        sal = lax.dot_general(s, bt, (((1,), (1,)), ((), ())),
                              preferred_element_type=jnp.float32)
        out_ref[pl.ds(g * 2 + bb, 1), :] = (
            jnp.max(sal, axis=1) / (C ** 0.5)).reshape(1, NS)


def _scores_call(sampled_tokens, base_tokens):
    return pl.pallas_call(
        _scores_body,
        grid=(B // 2,),
        in_specs=[
            pl.BlockSpec((2, NS, C), lambda g: (g, 0, 0)),
            pl.BlockSpec((2, NB, C), lambda g: (g, 0, 0)),
        ],
        out_specs=pl.BlockSpec((B, NS), lambda g: (0, 0)),
        out_shape=jax.ShapeDtypeStruct((B, NS), jnp.float32),
    )(sampled_tokens, base_tokens)


def _sc_topk_gather(scores, tokens_flat):
    mesh = plsc.VectorSubcoreMesh(core_axis_name="c", subcore_axis_name="s",
                                  num_cores=2, num_subcores=16)

    @functools.partial(
        pl.kernel,
        mesh=mesh,
        compiler_params=pltpu.CompilerParams(needs_layout_passes=False),
        out_type=[
            jax.ShapeDtypeStruct((B, TOPK), jnp.int32),
            jax.ShapeDtypeStruct((B, TOPK, C), jnp.float32),
        ],
        scratch_types=[
            pltpu.VMEM((NS,), jnp.float32),
            pltpu.VMEM((NS,), jnp.int32),
            pltpu.VMEM((NS + 16,), jnp.int32),
            pltpu.VMEM((TOPK,), jnp.int32),
            pltpu.VMEM((TOPK,), jnp.int32),
            pltpu.VMEM((TOPK,), jnp.int32),
            pltpu.VMEM((TOPK, C), jnp.float32),
            pltpu.SemaphoreType.DMA,
        ],
    )
    def body(scores_hbm, tok_hbm, idx_out, sel_out,
             sc_v, keys_v, win_v, wkey_v, ord_v, gidx_v, rows_v, sem):
        wid = lax.axis_index("s") * 2 + lax.axis_index("c")

        @pl.when(wid < B)
        def _():
            b = wid
            pltpu.sync_copy(scores_hbm.at[b], sc_v)

            iota16 = lax.broadcasted_iota(jnp.int32, (16,), 0)

            def key_body(j, _):
                for u in range(4):
                    v = sc_v[pl.ds(j * 64 + u * 16, 16)]
                    i = plsc.bitcast(v, jnp.int32)
                    keys_v[pl.ds(j * 64 + u * 16, 16)] = jnp.where(
                        i >= 0, i, i ^ jnp.int32(0x7FFFFFFF))
                return 0
            lax.fori_loop(0, NV // 4, key_body, 0)

            def count_ge(t):
                tv = jnp.full((16,), t, jnp.int32)
                def cb(j, cnt):
                    for u in range(8):
                        kv = keys_v[pl.ds(j * 128 + u * 16, 16)]
                        cnt = cnt + jnp.where(kv >= tv, 1, 0).astype(jnp.int32)
                    return cnt
                cnt = lax.fori_loop(0, NV // 8, cb,
                                    jnp.zeros((16,), jnp.int32))
                return jnp.sum(cnt)

            def bs_body(_, lh):
                lo, hi = lh
                x = lo ^ hi
                mid = (lo & hi) + (x >> 1) + (x & 1)
                ok = count_ge(mid) >= TOPK
                return (jnp.where(ok, mid, lo),
                        jnp.where(ok, hi, mid - 1))
            thr, _hi = lax.fori_loop(
                0, 32, bs_body,
                (jnp.int32(-(2 ** 31)), jnp.int32(2 ** 31 - 1)))
            thrv = jnp.full((16,), thr, jnp.int32)

            def comp(j, off, strict):
                for u in range(2):
                    kv = keys_v[pl.ds(j * 32 + u * 16, 16)]
                    m = (kv > thrv) if strict else (kv == thrv)
                    mi = jnp.where(m, 1, 0).astype(jnp.int32)
                    pos = off + plsc.cumsum(mi) - 1
                    plsc.store_scatter(win_v, [pos], iota16 + j * 32 + u * 16,
                                       mask=m)
                    off = off + jnp.sum(mi)
                return off
            off = lax.fori_loop(0, NV // 2, lambda j, o: comp(j, o, True),
                                jnp.int32(0))
            lax.fori_loop(0, NV // 2, lambda j, o: comp(j, o, False), off)

            wks = []
            wis = []
            for jj in range(TOPK // 16):
                wi = win_v[pl.ds(jj * 16, 16)]
                wk = plsc.load_gather(keys_v, [wi])
                wkey_v[pl.ds(jj * 16, 16)] = wk
                wks.append(wk)
                wis.append(wi)

            def rank_body(p, ranks):
                for u in range(2):
                    pv = jnp.full((16,), p * 2 + u, jnp.int32)
                    kb = plsc.load_gather(wkey_v, [pv])
                    ib = plsc.load_gather(win_v, [pv])
                    out = []
                    for i in range(TOPK // 16):
                        beat = (kb > wks[i]) | ((kb == wks[i]) & (ib < wis[i]))
                        out.append(ranks[i] +
                                   jnp.where(beat, 1, 0).astype(jnp.int32))
                    ranks = tuple(out)
                return ranks
            ranks = lax.fori_loop(
                0, TOPK // 2, rank_body,
                tuple(jnp.zeros((16,), jnp.int32) for _ in range(TOPK // 16)))

            for i in range(TOPK // 16):
                plsc.store_scatter(ord_v, [ranks[i]], wis[i])

            pltpu.sync_copy(ord_v, idx_out.at[b])

            def gidx_body(j, _):
                gidx_v[pl.ds(j * 16, 16)] = (
                    ord_v[pl.ds(j * 16, 16)] + b * NS)
                return 0
            lax.fori_loop(0, TOPK // 16, gidx_body, 0)

            pltpu.async_copy(tok_hbm.at[gidx_v], rows_v, sem).wait()
            pltpu.sync_copy(rows_v, sel_out.at[b])

    return body(scores, tokens_flat)


def _qproj_body(base_ref, wq_ref, bq_ref, q_out_ref, base16_ref):
    bf = jnp.bfloat16
    xb16 = base_ref[...].reshape(2 * NB, C).astype(bf)
    base16_ref[...] = xb16.reshape(2, NB, C)
    q = jnp.dot(xb16, wq_ref[...].astype(bf),
                preferred_element_type=jnp.float32) + bq_ref[...]
    q_out_ref[...] = q.astype(bf).reshape(2, NB, C)


def _qproj_call(base_tokens, Wq, bq):
    return pl.pallas_call(
        _qproj_body,
        grid=(B // 2,),
        in_specs=[
            pl.BlockSpec((2, NB, C), lambda g: (g, 0, 0)),
            pl.BlockSpec((C, C), lambda g: (0, 0)),
            pl.BlockSpec((C,), lambda g: (0,)),
        ],
        out_specs=[
            pl.BlockSpec((2, NB, C), lambda g: (g, 0, 0)),
            pl.BlockSpec((2, NB, C), lambda g: (g, 0, 0)),
        ],
        out_shape=[
            jax.ShapeDtypeStruct((B, NB, C), jnp.bfloat16),
            jax.ShapeDtypeStruct((B, NB, C), jnp.bfloat16),
        ],
    )(base_tokens, Wq, bq)


def _attn_body(base_ref, sel_ref, q_ref, wk_ref, bk_ref, wv_ref,
               bv_ref, wo_ref, bo_ref, g1_ref, beta1_ref, x_ref, attn_ref,
               wk16_ref, wv16_ref, wo16_ref):
    bf = jnp.bfloat16

    @pl.when(pl.program_id(0) == 0)
    def _():
        wk16_ref[...] = wk_ref[...].astype(bf)
        wv16_ref[...] = wv_ref[...].astype(bf)
        wo16_ref[...] = wo_ref[...].astype(bf)

    for bb in range(1):
        xb = base_ref[bb].astype(jnp.float32)
        sel16 = sel_ref[bb].astype(bf)
        q16 = q_ref[bb]
        k = jnp.dot(sel16, wk16_ref[...],
                    preferred_element_type=jnp.float32) + bk_ref[...]
        v = jnp.dot(sel16, wv16_ref[...],
                    preferred_element_type=jnp.float32) + bv_ref[...]
        v16 = v.astype(bf)
        kt16 = jnp.transpose(k.astype(bf))
        att_cols = []
        for h in range(H):
            qh = q16[:, h * DH:(h + 1) * DH]
            kht = kt16[h * DH:(h + 1) * DH, :]
            vh = v16[:, h * DH:(h + 1) * DH]
            lg = jnp.dot(qh, kht,
                         preferred_element_type=jnp.float32) * (1.0 / DH ** 0.5)
            p = jnp.exp(lg)
            a = p * (1.0 / jnp.sum(p, axis=-1, keepdims=True))
            attn_ref[bb, h] = a
            att_cols.append(jnp.dot(a.astype(bf), vh,
                                    preferred_element_type=jnp.float32))
        attended = jnp.concatenate(att_cols, axis=1)
        o = jnp.dot(attended.astype(bf), wo16_ref[...],
                    preferred_element_type=jnp.float32) + bo_ref[...]
        y = xb + o
        mu = jnp.mean(y, axis=-1, keepdims=True)
        var = jnp.mean((y - mu) ** 2, axis=-1, keepdims=True)
        x = (y - mu) * lax.rsqrt(var + 1e-5) * g1_ref[...] + beta1_ref[...]
        x_ref[bb] = x.astype(bf)


def _attn_call(base_tokens, selected, q, Wk, bk, Wv, bv, Wo, bo, g1, beta1):
    w2 = lambda: pl.BlockSpec((C, C), lambda b: (0, 0))
    w1 = lambda: pl.BlockSpec((C,), lambda b: (0,))
    return pl.pallas_call(
        _attn_body,
        grid=(B,),
        in_specs=[
            pl.BlockSpec((1, NB, C), lambda g: (g, 0, 0)),
            pl.BlockSpec((1, TOPK, C), lambda g: (g, 0, 0)),
            pl.BlockSpec((1, NB, C), lambda g: (g, 0, 0)),
            w2(), w1(), w2(), w1(), w2(), w1(), w1(), w1(),
        ],
        out_specs=[
            pl.BlockSpec((1, NB, C), lambda g: (g, 0, 0)),
            pl.BlockSpec((1, H, NB, TOPK), lambda g: (g, 0, 0, 0)),
        ],
        out_shape=[
            jax.ShapeDtypeStruct((B, NB, C), jnp.bfloat16),
            jax.ShapeDtypeStruct((B, H, NB, TOPK), jnp.float32),
        ],
        scratch_shapes=[
            pltpu.VMEM((C, C), jnp.bfloat16),
            pltpu.VMEM((C, C), jnp.bfloat16),
            pltpu.VMEM((C, C), jnp.bfloat16),
        ],
    )(base_tokens, selected, q, Wk, bk, Wv, bv, Wo, bo, g1, beta1)


def _mlp_body(x_ref, wm1_ref, bm1_ref, wm2_ref, bm2_ref, g2_ref, beta2_ref,
              out_ref, w116_ref, w216_ref):
    bf = jnp.bfloat16

    @pl.when(pl.program_id(0) == 0)
    def _():
        w116_ref[...] = wm1_ref[...].astype(bf)
        w216_ref[...] = wm2_ref[...].astype(bf)

    x16 = x_ref[...].reshape(2 * NB, C)
    x = x16.astype(jnp.float32)
    hmid = jnp.dot(x16, w116_ref[...],
                   preferred_element_type=jnp.float32) + bm1_ref[...]
    hmid = 0.5 * hmid * (1.0 + lax.erf(hmid * (2.0 ** -0.5)))
    mlp = jnp.dot(hmid.astype(bf), w216_ref[...],
                  preferred_element_type=jnp.float32)
    y = x + mlp + bm2_ref[...]
    mu = jnp.mean(y, axis=-1, keepdims=True)
    var = jnp.mean((y - mu) ** 2, axis=-1, keepdims=True)
    res = (y - mu) * lax.rsqrt(var + 1e-5) * g2_ref[...] + beta2_ref[...]
    out_ref[...] = res.reshape(2, NB, C)


def _mlp_call(x, Wm1, bm1, Wm2, bm2, g2, beta2):
    return pl.pallas_call(
        _mlp_body,
        grid=(B // 2,),
        in_specs=[
            pl.BlockSpec((2, NB, C), lambda g: (g, 0, 0)),
            pl.BlockSpec((C, HID), lambda g: (0, 0)),
            pl.BlockSpec((HID,), lambda g: (0,)),
            pl.BlockSpec((HID, C), lambda g: (0, 0)),
            pl.BlockSpec((C,), lambda g: (0,)),
            pl.BlockSpec((C,), lambda g: (0,)),
            pl.BlockSpec((C,), lambda g: (0,)),
        ],
        out_specs=pl.BlockSpec((2, NB, C), lambda g: (g, 0, 0)),
        out_shape=jax.ShapeDtypeStruct((B, NB, C), jnp.float32),
        scratch_shapes=[
            pltpu.VMEM((C, HID), jnp.bfloat16),
            pltpu.VMEM((HID, C), jnp.bfloat16),
        ],
    )(x, Wm1, bm1, Wm2, bm2, g2, beta2)


def kernel(base_tokens, sampled_tokens, Wq, bq, Wk, bk, Wv, bv, Wo, bo,
           g1, beta1, g2, beta2, Wm1, bm1, Wm2, bm2):
    scores = _scores_call(sampled_tokens, base_tokens)
    topk_idx, selected = _sc_topk_gather(
        scores, sampled_tokens.reshape(B * NS, C))
    q, base16 = _qproj_call(base_tokens, Wq, bq)
    x1, attn = _attn_call(base16, selected, q, Wk, bk, Wv, bv,
                          Wo, bo, g1, beta1)
    xf = _mlp_call(x1, Wm1, bm1, Wm2, bm2, g2, beta2)
    return (xf, scores, attn, topk_idx, selected)


# speedup vs baseline: 1.0033x; 1.0033x over previous
import functools
import math

import jax
import jax.numpy as jnp
from jax import lax
from jax.experimental import pallas as pl
from jax.experimental.pallas import tpu as pltpu
from jax.experimental.pallas import tpu_sc as plsc

B, NB, NS, C, H, TOPK = 8, 576, 1024, 768, 12, 128
DH = C // H
HID = 4 * C
NV = NS // 16


def _scores_body(sampled_ref, base_ref, out_ref):
    g = pl.program_id(0)
    for bb in range(2):
        s = sampled_ref[bb]
        bt = base_ref[bb]
        sal = lax.dot_general(s, bt, (((1,), (1,)), ((), ())),
                              preferred_element_type=jnp.float32)
        out_ref[pl.ds(g * 2 + bb, 1), :] = (
            jnp.max(sal, axis=1) / (C ** 0.5)).reshape(1, NS)


def _scores_call(sampled_tokens, base_tokens):
    return pl.pallas_call(
        _scores_body,
        grid=(B // 2,),
        in_specs=[
            pl.BlockSpec((2, NS, C), lambda g: (g, 0, 0)),
            pl.BlockSpec((2, NB, C), lambda g: (g, 0, 0)),
        ],
        out_specs=pl.BlockSpec((B, NS), lambda g: (0, 0)),
        out_shape=jax.ShapeDtypeStruct((B, NS), jnp.float32),
    )(sampled_tokens, base_tokens)


def _sc_topk_gather(scores, tokens_flat):
    mesh = plsc.VectorSubcoreMesh(core_axis_name="c", subcore_axis_name="s",
                                  num_cores=2, num_subcores=16)

    @functools.partial(
        pl.kernel,
        mesh=mesh,
        compiler_params=pltpu.CompilerParams(needs_layout_passes=False),
        out_type=[
            jax.ShapeDtypeStruct((B, TOPK), jnp.int32),
            jax.ShapeDtypeStruct((B, TOPK, C), jnp.float32),
        ],
        scratch_types=[
            pltpu.VMEM((NS,), jnp.float32),
            pltpu.VMEM((NS,), jnp.int32),
            pltpu.VMEM((NS + 16,), jnp.int32),
            pltpu.VMEM((TOPK,), jnp.int32),
            pltpu.VMEM((TOPK,), jnp.int32),
            pltpu.VMEM((TOPK,), jnp.int32),
            pltpu.VMEM((TOPK, C), jnp.float32),
            pltpu.SemaphoreType.DMA,
        ],
    )
    def body(scores_hbm, tok_hbm, idx_out, sel_out,
             sc_v, keys_v, win_v, wkey_v, ord_v, gidx_v, rows_v, sem):
        wid = lax.axis_index("s") * 2 + lax.axis_index("c")

        @pl.when(wid < B)
        def _():
            b = wid
            pltpu.sync_copy(scores_hbm.at[b], sc_v)

            iota16 = lax.broadcasted_iota(jnp.int32, (16,), 0)

            def key_body(j, _):
                for u in range(4):
                    v = sc_v[pl.ds(j * 64 + u * 16, 16)]
                    i = plsc.bitcast(v, jnp.int32)
                    keys_v[pl.ds(j * 64 + u * 16, 16)] = jnp.where(
                        i >= 0, i, i ^ jnp.int32(0x7FFFFFFF))
                return 0
            lax.fori_loop(0, NV // 4, key_body, 0)

            def count_ge(t):
                tv = jnp.full((16,), t, jnp.int32)
                def cb(j, cnt):
                    for u in range(8):
                        kv = keys_v[pl.ds(j * 128 + u * 16, 16)]
                        cnt = cnt + jnp.where(kv >= tv, 1, 0).astype(jnp.int32)
                    return cnt
                cnt = lax.fori_loop(0, NV // 8, cb,
                                    jnp.zeros((16,), jnp.int32))
                return jnp.sum(cnt)

            def bs_body(_, lh):
                lo, hi = lh
                x = lo ^ hi
                mid = (lo & hi) + (x >> 1) + (x & 1)
                ok = count_ge(mid) >= TOPK
                return (jnp.where(ok, mid, lo),
                        jnp.where(ok, hi, mid - 1))
            thr, _hi = lax.fori_loop(
                0, 32, bs_body,
                (jnp.int32(-(2 ** 31)), jnp.int32(2 ** 31 - 1)))
            thrv = jnp.full((16,), thr, jnp.int32)

            def comp(j, off, strict):
                for u in range(2):
                    kv = keys_v[pl.ds(j * 32 + u * 16, 16)]
                    m = (kv > thrv) if strict else (kv == thrv)
                    mi = jnp.where(m, 1, 0).astype(jnp.int32)
                    pos = off + plsc.cumsum(mi) - 1
                    plsc.store_scatter(win_v, [pos], iota16 + j * 32 + u * 16,
                                       mask=m)
                    off = off + jnp.sum(mi)
                return off
            off = lax.fori_loop(0, NV // 2, lambda j, o: comp(j, o, True),
                                jnp.int32(0))
            lax.fori_loop(0, NV // 2, lambda j, o: comp(j, o, False), off)

            wks = []
            wis = []
            for jj in range(TOPK // 16):
                wi = win_v[pl.ds(jj * 16, 16)]
                wk = plsc.load_gather(keys_v, [wi])
                wkey_v[pl.ds(jj * 16, 16)] = wk
                wks.append(wk)
                wis.append(wi)

            def rank_body(p, ranks):
                for u in range(2):
                    pv = jnp.full((16,), p * 2 + u, jnp.int32)
                    kb = plsc.load_gather(wkey_v, [pv])
                    ib = plsc.load_gather(win_v, [pv])
                    out = []
                    for i in range(TOPK // 16):
                        beat = (kb > wks[i]) | ((kb == wks[i]) & (ib < wis[i]))
                        out.append(ranks[i] +
                                   jnp.where(beat, 1, 0).astype(jnp.int32))
                    ranks = tuple(out)
                return ranks
            ranks = lax.fori_loop(
                0, TOPK // 2, rank_body,
                tuple(jnp.zeros((16,), jnp.int32) for _ in range(TOPK // 16)))

            for i in range(TOPK // 16):
                plsc.store_scatter(ord_v, [ranks[i]], wis[i])

            pltpu.sync_copy(ord_v, idx_out.at[b])

            def gidx_body(j, _):
                gidx_v[pl.ds(j * 16, 16)] = (
                    ord_v[pl.ds(j * 16, 16)] + b * NS)
                return 0
            lax.fori_loop(0, TOPK // 16, gidx_body, 0)

            pltpu.async_copy(tok_hbm.at[gidx_v], rows_v, sem).wait()
            pltpu.sync_copy(rows_v, sel_out.at[b])

    return body(scores, tokens_flat)


def _qproj_body(base_ref, wq_ref, bq_ref, q_out_ref):
    bf = jnp.bfloat16
    xb16 = base_ref[...].reshape(2 * NB, C).astype(bf)
    q = jnp.dot(xb16, wq_ref[...].astype(bf),
                preferred_element_type=jnp.float32) + bq_ref[...]
    q_out_ref[...] = q.astype(bf).reshape(2, NB, C)


def _qproj_call(base_tokens, Wq, bq):
    return pl.pallas_call(
        _qproj_body,
        grid=(B // 2,),
        in_specs=[
            pl.BlockSpec((2, NB, C), lambda g: (g, 0, 0)),
            pl.BlockSpec((C, C), lambda g: (0, 0)),
            pl.BlockSpec((C,), lambda g: (0,)),
        ],
        out_specs=pl.BlockSpec((2, NB, C), lambda g: (g, 0, 0)),
        out_shape=jax.ShapeDtypeStruct((B, NB, C), jnp.bfloat16),
    )(base_tokens, Wq, bq)


def _attn_body(base_ref, sel_ref, q_ref, wk_ref, bk_ref, wv_ref,
               bv_ref, wo_ref, bo_ref, g1_ref, beta1_ref, x_ref, attn_ref,
               wk16_ref, wv16_ref, wo16_ref):
    bf = jnp.bfloat16

    @pl.when(pl.program_id(0) == 0)
    def _():
        wk16_ref[...] = wk_ref[...].astype(bf)
        wv16_ref[...] = wv_ref[...].astype(bf)
        wo16_ref[...] = wo_ref[...].astype(bf)

    for bb in range(1):
        xb = base_ref[bb]
        sel16 = sel_ref[bb].astype(bf)
        q16 = q_ref[bb]
        k = jnp.dot(sel16, wk16_ref[...],
                    preferred_element_type=jnp.float32) + bk_ref[...]
        v = jnp.dot(sel16, wv16_ref[...],
                    preferred_element_type=jnp.float32) + bv_ref[...]
        v16 = v.astype(bf)
        kt16 = jnp.transpose(k.astype(bf))
        att_cols = []
        for h in range(H):
            qh = q16[:, h * DH:(h + 1) * DH]
            kht = kt16[h * DH:(h + 1) * DH, :]
            vh = v16[:, h * DH:(h + 1) * DH]
            lg = jnp.dot(qh, kht,
                         preferred_element_type=jnp.float32) * (1.0 / DH ** 0.5)
            p = jnp.exp(lg)
            a = p * (1.0 / jnp.sum(p, axis=-1, keepdims=True))
            attn_ref[bb, h] = a
            att_cols.append(jnp.dot(a.astype(bf), vh,
                                    preferred_element_type=jnp.float32))
        attended = jnp.concatenate(att_cols, axis=1)
        o = jnp.dot(attended.astype(bf), wo16_ref[...],
                    preferred_element_type=jnp.float32) + bo_ref[...]
        y = xb + o
        mu = jnp.mean(y, axis=-1, keepdims=True)
        var = jnp.mean((y - mu) ** 2, axis=-1, keepdims=True)
        x = (y - mu) * lax.rsqrt(var + 1e-5) * g1_ref[...] + beta1_ref[...]
        x_ref[bb] = x.astype(bf)


def _attn_call(base_tokens, selected, q, Wk, bk, Wv, bv, Wo, bo, g1, beta1):
    w2 = lambda: pl.BlockSpec((C, C), lambda b: (0, 0))
    w1 = lambda: pl.BlockSpec((C,), lambda b: (0,))
    return pl.pallas_call(
        _attn_body,
        grid=(B,),
        in_specs=[
            pl.BlockSpec((1, NB, C), lambda g: (g, 0, 0)),
            pl.BlockSpec((1, TOPK, C), lambda g: (g, 0, 0)),
            pl.BlockSpec((1, NB, C), lambda g: (g, 0, 0)),
            w2(), w1(), w2(), w1(), w2(), w1(), w1(), w1(),
        ],
        out_specs=[
            pl.BlockSpec((1, NB, C), lambda g: (g, 0, 0)),
            pl.BlockSpec((1, H, NB, TOPK), lambda g: (g, 0, 0, 0)),
        ],
        out_shape=[
            jax.ShapeDtypeStruct((B, NB, C), jnp.bfloat16),
            jax.ShapeDtypeStruct((B, H, NB, TOPK), jnp.float32),
        ],
        scratch_shapes=[
            pltpu.VMEM((C, C), jnp.bfloat16),
            pltpu.VMEM((C, C), jnp.bfloat16),
            pltpu.VMEM((C, C), jnp.bfloat16),
        ],
    )(base_tokens, selected, q, Wk, bk, Wv, bv, Wo, bo, g1, beta1)


def _mlp_body(x_ref, wm1_ref, bm1_ref, wm2_ref, bm2_ref, g2_ref, beta2_ref,
              out_ref, w116_ref, w216_ref):
    bf = jnp.bfloat16

    @pl.when(pl.program_id(0) == 0)
    def _():
        w116_ref[...] = wm1_ref[...].astype(bf)
        w216_ref[...] = wm2_ref[...].astype(bf)

    x16 = x_ref[...].reshape(2 * NB, C)
    x = x16.astype(jnp.float32)
    hmid = jnp.dot(x16, w116_ref[...],
                   preferred_element_type=jnp.float32) + bm1_ref[...]
    hmid = 0.5 * hmid * (1.0 + lax.erf(hmid * (2.0 ** -0.5)))
    mlp = jnp.dot(hmid.astype(bf), w216_ref[...],
                  preferred_element_type=jnp.float32)
    y = x + mlp + bm2_ref[...]
    mu = jnp.mean(y, axis=-1, keepdims=True)
    var = jnp.mean((y - mu) ** 2, axis=-1, keepdims=True)
    res = (y - mu) * lax.rsqrt(var + 1e-5) * g2_ref[...] + beta2_ref[...]
    out_ref[...] = res.reshape(2, NB, C)


def _mlp_call(x, Wm1, bm1, Wm2, bm2, g2, beta2):
    return pl.pallas_call(
        _mlp_body,
        grid=(B // 2,),
        in_specs=[
            pl.BlockSpec((2, NB, C), lambda g: (g, 0, 0)),
            pl.BlockSpec((C, HID), lambda g: (0, 0)),
            pl.BlockSpec((HID,), lambda g: (0,)),
            pl.BlockSpec((HID, C), lambda g: (0, 0)),
            pl.BlockSpec((C,), lambda g: (0,)),
            pl.BlockSpec((C,), lambda g: (0,)),
            pl.BlockSpec((C,), lambda g: (0,)),
        ],
        out_specs=pl.BlockSpec((2, NB, C), lambda g: (g, 0, 0)),
        out_shape=jax.ShapeDtypeStruct((B, NB, C), jnp.float32),
        scratch_shapes=[
            pltpu.VMEM((C, HID), jnp.bfloat16),
            pltpu.VMEM((HID, C), jnp.bfloat16),
        ],
    )(x, Wm1, bm1, Wm2, bm2, g2, beta2)


def kernel(base_tokens, sampled_tokens, Wq, bq, Wk, bk, Wv, bv, Wo, bo,
           g1, beta1, g2, beta2, Wm1, bm1, Wm2, bm2):
    scores = _scores_call(sampled_tokens, base_tokens)
    topk_idx, selected = _sc_topk_gather(
        scores, sampled_tokens.reshape(B * NS, C))
    q = _qproj_call(base_tokens, Wq, bq)
    x1, attn = _attn_call(base_tokens, selected, q, Wk, bk, Wv, bv,
                          Wo, bo, g1, beta1)
    xf = _mlp_call(x1, Wm1, bm1, Wm2, bm2, g2, beta2)
    return (xf, scores, attn, topk_idx, selected)
